# baseline (speedup 1.0000x reference)
import functools

import jax
import jax.numpy as jnp
from jax import lax
from jax.experimental import pallas as pl
from jax.experimental.pallas import tpu as pltpu
from jax.experimental.pallas import tpu_sc as plsc

N = 10000
E = 160000
B = 64
H = 1024

NP = 10240
EE = E + N
CH = 512
EP = EE + CH
NW = 32
RPW = 40
NB = 32
NPASS = NP // (NW * RPW)
BM = 512
GRID = NP // BM


def _ln(x, g, b):
    m = jnp.mean(x, axis=-1, keepdims=True)
    xc = x - m
    v = jnp.mean(xc * xc, axis=-1, keepdims=True)
    return xc * lax.rsqrt(v + 1e-5) * g + b


def _mm0_body(x_ref, w_ref, a_ref, h_ref, asad_ref):
    h = jnp.dot(x_ref[...], w_ref[...], preferred_element_type=jnp.float32)
    h_ref[...] = h
    asad_ref[...] = jnp.dot(h, a_ref[...], preferred_element_type=jnp.float32)


def _mm_body(g_ref, bias_ref, gam_ref, bet_ref, w_ref, a_ref, h_ref, asad_ref):
    x = g_ref[...] + bias_ref[...]
    x = jnp.maximum(_ln(x, gam_ref[...], bet_ref[...]), 0.0)
    h = jnp.dot(x, w_ref[...], preferred_element_type=jnp.float32)
    h_ref[...] = h
    asad_ref[...] = jnp.dot(h, a_ref[...], preferred_element_type=jnp.float32)


def _final_body(g_ref, bias_ref, gam_ref, bet_ref, gw_ref, gb_ref,
                x_ref, gate_ref):
    x = g_ref[...] + bias_ref[...]
    x = jnp.maximum(_ln(x, gam_ref[...], bet_ref[...]), 0.0)
    x_ref[...] = x
    z = jnp.dot(x, gw_ref[...], preferred_element_type=jnp.float32) + gb_ref[...]
    gate_ref[...] = jax.nn.sigmoid(z)


def _pool_body(x_ref, gate_ref, batch_ref, u_ref, den_ref):
    i = pl.program_id(0)

    @pl.when(i == 0)
    def _():
        u_ref[...] = jnp.zeros_like(u_ref)
        den_ref[...] = jnp.zeros_like(den_ref)

    bvals = batch_ref[0]
    ge = jnp.exp(gate_ref[0])
    sel = bvals == lax.broadcasted_iota(jnp.int32, (B, BM), 0)
    w = jnp.where(sel, jnp.broadcast_to(ge, (B, BM)), 0.0)
    u_ref[...] += jnp.dot(w, x_ref[...], preferred_element_type=jnp.float32)
    s = jnp.sum(w, axis=1, keepdims=True)
    den_ref[...] += jnp.broadcast_to(s, (B, 128))


def _mlp_body(u_ref, den_ref,
              w0, b0, g0, t0, w1, b1, g1, t1, w2, b2, g2, t2,
              w3, b3, g3, t3, w4, b4, g4, t4, ow, ob, out_ref):
    y = u_ref[...] / (den_ref[...][:, :1] + 1e-16)
    for w_r, b_r, g_r, t_r in ((w0, b0, g0, t0), (w1, b1, g1, t1),
                               (w2, b2, g2, t2), (w3, b3, g3, t3),
                               (w4, b4, g4, t4)):
        y = jnp.dot(y, w_r[...], preferred_element_type=jnp.float32) + b_r[...]
        y = jnp.maximum(_ln(y, g_r[...], t_r[...]), 0.0)
    out_ref[...] = (jnp.dot(y, ow[...], preferred_element_type=jnp.float32)
                    + ob[...])


def _row_spec(last):
    return pl.BlockSpec((BM, last), lambda i: (i, 0))


def _full_spec(shape):
    return pl.BlockSpec(shape, lambda i: tuple(0 for _ in shape))


def _gat_mm0(x, w, a):
    return pl.pallas_call(
        _mm0_body,
        grid=(GRID,),
        in_specs=[_row_spec(32), _full_spec((32, H)), _full_spec((H, 128))],
        out_specs=[_row_spec(H), _row_spec(128)],
        out_shape=[jax.ShapeDtypeStruct((NP, H), jnp.float32),
                   jax.ShapeDtypeStruct((NP, 128), jnp.float32)],
    )(x, w, a)


def _gat_mm(g, bias, gam, bet, w, a):
    return pl.pallas_call(
        _mm_body,
        grid=(GRID,),
        in_specs=[_row_spec(H), _full_spec((1, H)), _full_spec((1, H)),
                  _full_spec((1, H)), _full_spec((H, H)), _full_spec((H, 128))],
        out_specs=[_row_spec(H), _row_spec(128)],
        out_shape=[jax.ShapeDtypeStruct((NP, H), jnp.float32),
                   jax.ShapeDtypeStruct((NP, 128), jnp.float32)],
    )(g, bias, gam, bet, w, a)


def _final_tc(g, bias, gam, bet, gw, gb):
    return pl.pallas_call(
        _final_body,
        grid=(GRID,),
        in_specs=[_row_spec(H), _full_spec((1, H)), _full_spec((1, H)),
                  _full_spec((1, H)), _full_spec((H, 128)), _full_spec((1, 128))],
        out_specs=[_row_spec(H), _row_spec(128)],
        out_shape=[jax.ShapeDtypeStruct((NP, H), jnp.float32),
                   jax.ShapeDtypeStruct((NP, 128), jnp.float32)],
    )(g, bias, gam, bet, gw, gb)


def _pool_tc(x, gate3, batch3):
    return pl.pallas_call(
        _pool_body,
        grid=(GRID,),
        in_specs=[_row_spec(H),
                  pl.BlockSpec((1, 1, BM), lambda i: (i, 0, 0)),
                  pl.BlockSpec((1, 1, BM), lambda i: (i, 0, 0))],
        out_specs=[_full_spec((B, H)), _full_spec((B, 128))],
        out_shape=[jax.ShapeDtypeStruct((B, H), jnp.float32),
                   jax.ShapeDtypeStruct((B, 128), jnp.float32)],
    )(x, gate3, batch3)


def _mlp_tc(u, den, layers, ow, ob):
    flat = []
    for lw, lb, lg, lt in layers:
        flat += [lw, lb, lg, lt]
    specs = [_full_spec((B, H)), _full_spec((B, 128))]
    for _ in range(5):
        specs += [_full_spec((H, H)), _full_spec((1, H)),
                  _full_spec((1, H)), _full_spec((1, H))]
    specs += [_full_spec((H, 128)), _full_spec((1, 128))]
    return pl.pallas_call(
        _mlp_body,
        grid=(1,),
        in_specs=specs,
        out_specs=[_full_spec((B, 128))],
        out_shape=[jax.ShapeDtypeStruct((B, 128), jnp.float32)],
    )(u, den, *flat, ow, ob)


def _edge_body(h_hbm, asv_hbm, adv_hbm, src_hbm, dst_hbm, rp_hbm, z_hbm,
               out_hbm,
               acc, stage_a, stage_b, srcb, dstb, asb, adb, rpb, den, albuf,
               sem_a, sem_b, sem_ra, sem_rb):
    wid = lax.axis_index("s") * 2 + lax.axis_index("c")
    wid = pl.multiple_of(wid, 1)

    def edge_vals(t, c0, e_lo, e_hi, base):
        sl = pl.ds(t * 16, 16)
        gid = c0 + t * 16 + lax.iota(jnp.int32, 16)
        mask = (gid >= e_lo) & (gid < e_hi)
        s = asb[sl] + adb[sl]
        e = jnp.maximum(s, 0.2 * s)
        ee = jnp.exp(jnp.minimum(e, 60.0))
        eem = jnp.where(mask, ee, 0.0)
        dl = jnp.clip(dstb[sl] - base, 0, RPW - 1)
        return eem, dl

    def load_chunk(c0):
        c0 = pl.multiple_of(c0, 8)
        cp1 = pltpu.make_async_copy(src_hbm.at[pl.ds(c0, CH)], srcb, sem_a)
        cp2 = pltpu.make_async_copy(dst_hbm.at[pl.ds(c0, CH)], dstb, sem_b)
        cp1.start(); cp2.start()
        cp1.wait(); cp2.wait()
        gathers = []
        for q in range(CH // 128):
            qs = pl.ds(q * 128, 128)
            gathers.append(pltpu.make_async_copy(
                asv_hbm.at[srcb.at[qs]], asb.at[qs], sem_a))
            gathers.append(pltpu.make_async_copy(
                adv_hbm.at[dstb.at[qs]], adb.at[qs], sem_b))
        for cp in gathers:
            cp.start()
        for cp in gathers:
            cp.wait()

    def one_pass(p, pcarry):
        base = pl.multiple_of((p * NW + wid) * RPW, RPW)

        pltpu.sync_copy(z_hbm, acc)
        for q in range(3):
            den[pl.ds(q * 16, 16)] = jnp.zeros((16,), jnp.float32)

        pltpu.sync_copy(rp_hbm.at[pl.ds(base, 56)], rpb)
        e_lo = rpb[pl.ds(0, 16)][0]
        e_hi = rpb[pl.ds(RPW, 16)][0]
        e_al = e_lo - lax.rem(e_lo, 8)
        nch = lax.shift_right_logical(e_hi - e_al + (CH - 1), 9)

        def sweep1(c, _):
            c0 = e_al + c * CH
            load_chunk(c0)

            @plsc.parallel_loop(0, CH // 16, unroll=4)
            def _grp(t):
                eem, dl = edge_vals(t, c0, e_lo, e_hi, base)
                plsc.addupdate_scatter(den, [dl], eem)

            return _

        lax.fori_loop(0, nch, sweep1, 0)

        def fire(t, stage, sem):
            pltpu.make_async_copy(
                h_hbm.at[srcb.at[pl.ds(t * NB, NB)]], stage, sem).start()

        def proc(t, c0, stage, sem):
            dls = []
            for g in range(NB // 16):
                eem, dl = edge_vals(t * (NB // 16) + g, c0, e_lo, e_hi, base)
                denv = plsc.load_gather(den, [dl])
                albuf[pl.ds(g * 16, 16)] = eem / (denv + 1e-16)
                dls.append(dl)
            pltpu.make_async_copy(
                h_hbm.at[srcb.at[pl.ds(t * NB, NB)]], stage, sem).wait()

            for j in range(NB):
                ofs = dls[j // 16][j % 16] * H
                alv = plsc.load_gather(
                    albuf, [jnp.full((16,), j, jnp.int32)])

                @plsc.parallel_loop(0, H // 16, unroll=16)
                def _row(k):
                    k16 = k * 16
                    plsc.addupdate(acc.at[pl.ds(ofs + k16, 16)],
                                   alv * stage[j, pl.ds(k16, 16)])

        NBCH = CH // NB

        def sweep2(c, _):
            c0 = e_al + c * CH
            load_chunk(c0)
            fire(0, stage_a, sem_ra)

            def pair(i, carry):
                t0 = 2 * i
                fire(t0 + 1, stage_b, sem_rb)
                proc(t0, c0, stage_a, sem_ra)

                @pl.when(i < NBCH // 2 - 1)
                def _prefetch():
                    fire(t0 + 2, stage_a, sem_ra)

                proc(t0 + 1, c0, stage_b, sem_rb)
                return carry

            lax.fori_loop(0, NBCH // 2, pair, 0)
            return _

        lax.fori_loop(0, nch, sweep2, 0)

        pltpu.sync_copy(acc, out_hbm.at[pl.ds(pl.multiple_of(base * H, RPW * H),
                                              RPW * H)])
        return pcarry

    lax.fori_loop(0, NPASS, one_pass, 0)


def _edge_sc(h, asv, adv, srcs, dsts, rowptr, zeros):
    mesh = plsc.VectorSubcoreMesh(core_axis_name="c", subcore_axis_name="s",
                                  num_cores=2, num_subcores=16)
    f = pl.kernel(
        _edge_body,
        out_type=jax.ShapeDtypeStruct((NP * H,), jnp.float32),
        mesh=mesh,
        scratch_types=[
            pltpu.VMEM((RPW * H,), jnp.float32),
            pltpu.VMEM((NB, H), jnp.float32),
            pltpu.VMEM((NB, H), jnp.float32),
            pltpu.VMEM((CH,), jnp.int32),
            pltpu.VMEM((CH,), jnp.int32),
            pltpu.VMEM((CH,), jnp.float32),
            pltpu.VMEM((CH,), jnp.float32),
            pltpu.VMEM((56,), jnp.int32),
            pltpu.VMEM((48,), jnp.float32),
            pltpu.VMEM((NB,), jnp.float32),
            pltpu.SemaphoreType.DMA,
            pltpu.SemaphoreType.DMA,
            pltpu.SemaphoreType.DMA,
            pltpu.SemaphoreType.DMA,
        ],
        compiler_params=pltpu.CompilerParams(needs_layout_passes=False),
    )
    return f(h, asv, adv, srcs, dsts, rowptr, zeros)


def kernel(tree_x, edge_index, batch, params):
    f32 = jnp.float32

    loops = jnp.arange(N, dtype=jnp.int32)
    src0 = jnp.concatenate([edge_index[0].astype(jnp.int32), loops])
    dst0 = jnp.concatenate([edge_index[1].astype(jnp.int32), loops])
    key = (dst0.astype(jnp.uint32) << 18) | jnp.arange(EE, dtype=jnp.uint32)
    skey = lax.sort(key, is_stable=False)
    srcs = jnp.zeros((EP,), jnp.int32).at[:EE].set(
        src0[(skey & 0x3FFFF).astype(jnp.int32)])
    dsts = jnp.zeros((EP,), jnp.int32).at[:EE].set(
        (skey >> 18).astype(jnp.int32))
    rowptr = jnp.searchsorted(dsts[:EE], jnp.arange(NP + 1, dtype=jnp.int32),
                              side="left").astype(jnp.int32)
    rowptr = jnp.pad(rowptr, (0, 10256 - (NP + 1)), constant_values=EE)

    batch_p = jnp.full((NP,), B, jnp.int32).at[:N].set(batch.astype(jnp.int32))
    batch3 = batch_p.reshape(GRID, 1, BM)

    x0 = jnp.zeros((NP, 32), f32).at[:N, :17].set(tree_x)
    zeros = jnp.zeros((RPW * H,), f32)

    def avec(i):
        a = jnp.zeros((H, 128), f32)
        a = a.at[:, 0].set(params[f"gat{i}_asrc"])
        return a.at[:, 1].set(params[f"gat{i}_adst"])

    def row(v):
        return v.reshape(1, -1).astype(f32)

    w0 = jnp.zeros((32, H), f32).at[:17].set(params["gat0_W"])
    h, asad = _gat_mm0(x0, w0, avec(0))
    g = _edge_sc(h, asad[:, 0], asad[:, 1], srcs, dsts, rowptr, zeros)
    g = g.reshape(NP, H)
    for i in range(1, 5):
        h, asad = _gat_mm(g, row(params[f"gat{i - 1}_b"]),
                          row(params[f"gnorm{i - 1}_g"]),
                          row(params[f"gnorm{i - 1}_b"]),
                          params[f"gat{i}_W"], avec(i))
        g = _edge_sc(h, asad[:, 0], asad[:, 1], srcs, dsts, rowptr, zeros)
        g = g.reshape(NP, H)

    gw = jnp.zeros((H, 128), f32).at[:, 0].set(params["gate_W"][:, 0])
    gb = jnp.zeros((1, 128), f32).at[0, 0].set(params["gate_b"][0])
    x5, gate = _final_tc(g, row(params["gat4_b"]), row(params["gnorm4_g"]),
                         row(params["gnorm4_b"]), gw, gb)
    gate3 = gate[:, 0].reshape(GRID, 1, BM)
    u, den = _pool_tc(x5, gate3, batch3)

    layers = [(params[f"fcn{i}_W"], row(params[f"fcn{i}_b"]),
               row(params[f"fnorm{i}_g"]), row(params[f"fnorm{i}_b"]))
              for i in range(5)]
    ow = jnp.zeros((H, 128), f32).at[:, 0].set(params["out_W"][:, 0])
    ob = jnp.zeros((1, 128), f32).at[0, 0].set(params["out_b"][0])
    out, = _mlp_tc(u, den, layers, ow, ob)
    return out[:, :1]


# speedup vs baseline: 1.1796x; 1.1796x over previous
import functools

import jax
import jax.numpy as jnp
from jax import lax
from jax.experimental import pallas as pl
from jax.experimental.pallas import tpu as pltpu
from jax.experimental.pallas import tpu_sc as plsc

N = 10000
E = 160000
B = 64
H = 1024

NP = 10240
EE = E + N
CH = 512
EP = EE + CH
NW = 32
RPW = 40
NB = 32
NPASS = NP // (NW * RPW)
BM = 512
GRID = NP // BM


def _ln(x, g, b):
    m = jnp.mean(x, axis=-1, keepdims=True)
    xc = x - m
    v = jnp.mean(xc * xc, axis=-1, keepdims=True)
    return xc * lax.rsqrt(v + 1e-5) * g + b


def _mm0_body(x_ref, w_ref, a_ref, h_ref, asad_ref):
    h = jnp.dot(x_ref[...], w_ref[...], preferred_element_type=jnp.float32)
    h_ref[...] = h
    asad_ref[...] = jnp.dot(h, a_ref[...], preferred_element_type=jnp.float32)


def _mm_body(g_ref, bias_ref, gam_ref, bet_ref, w_ref, a_ref, h_ref, asad_ref):
    x = g_ref[...] + bias_ref[...]
    x = jnp.maximum(_ln(x, gam_ref[...], bet_ref[...]), 0.0)
    h = jnp.dot(x, w_ref[...], preferred_element_type=jnp.float32)
    h_ref[...] = h
    asad_ref[...] = jnp.dot(h, a_ref[...], preferred_element_type=jnp.float32)


def _final_body(g_ref, bias_ref, gam_ref, bet_ref, gw_ref, gb_ref,
                x_ref, gate_ref):
    x = g_ref[...] + bias_ref[...]
    x = jnp.maximum(_ln(x, gam_ref[...], bet_ref[...]), 0.0)
    x_ref[...] = x
    z = jnp.dot(x, gw_ref[...], preferred_element_type=jnp.float32) + gb_ref[...]
    gate_ref[...] = jax.nn.sigmoid(z)


def _pool_body(x_ref, gate_ref, batch_ref, u_ref, den_ref):
    i = pl.program_id(0)

    @pl.when(i == 0)
    def _():
        u_ref[...] = jnp.zeros_like(u_ref)
        den_ref[...] = jnp.zeros_like(den_ref)

    bvals = batch_ref[0]
    ge = jnp.exp(gate_ref[0])
    sel = bvals == lax.broadcasted_iota(jnp.int32, (B, BM), 0)
    w = jnp.where(sel, jnp.broadcast_to(ge, (B, BM)), 0.0)
    u_ref[...] += jnp.dot(w, x_ref[...], preferred_element_type=jnp.float32)
    s = jnp.sum(w, axis=1, keepdims=True)
    den_ref[...] += jnp.broadcast_to(s, (B, 128))


def _mlp_body(u_ref, den_ref,
              w0, b0, g0, t0, w1, b1, g1, t1, w2, b2, g2, t2,
              w3, b3, g3, t3, w4, b4, g4, t4, ow, ob, out_ref):
    y = u_ref[...] / (den_ref[...][:, :1] + 1e-16)
    for w_r, b_r, g_r, t_r in ((w0, b0, g0, t0), (w1, b1, g1, t1),
                               (w2, b2, g2, t2), (w3, b3, g3, t3),
                               (w4, b4, g4, t4)):
        y = jnp.dot(y, w_r[...], preferred_element_type=jnp.float32) + b_r[...]
        y = jnp.maximum(_ln(y, g_r[...], t_r[...]), 0.0)
    out_ref[...] = (jnp.dot(y, ow[...], preferred_element_type=jnp.float32)
                    + ob[...])


def _row_spec(last):
    return pl.BlockSpec((BM, last), lambda i: (i, 0))


def _full_spec(shape):
    return pl.BlockSpec(shape, lambda i: tuple(0 for _ in shape))


def _gat_mm0(x, w, a):
    return pl.pallas_call(
        _mm0_body,
        grid=(GRID,),
        in_specs=[_row_spec(32), _full_spec((32, H)), _full_spec((H, 128))],
        out_specs=[_row_spec(H), _row_spec(128)],
        out_shape=[jax.ShapeDtypeStruct((NP, H), jnp.float32),
                   jax.ShapeDtypeStruct((NP, 128), jnp.float32)],
    )(x, w, a)


def _gat_mm(g, bias, gam, bet, w, a):
    return pl.pallas_call(
        _mm_body,
        grid=(GRID,),
        in_specs=[_row_spec(H), _full_spec((1, H)), _full_spec((1, H)),
                  _full_spec((1, H)), _full_spec((H, H)), _full_spec((H, 128))],
        out_specs=[_row_spec(H), _row_spec(128)],
        out_shape=[jax.ShapeDtypeStruct((NP, H), jnp.float32),
                   jax.ShapeDtypeStruct((NP, 128), jnp.float32)],
    )(g, bias, gam, bet, w, a)


def _final_tc(g, bias, gam, bet, gw, gb):
    return pl.pallas_call(
        _final_body,
        grid=(GRID,),
        in_specs=[_row_spec(H), _full_spec((1, H)), _full_spec((1, H)),
                  _full_spec((1, H)), _full_spec((H, 128)), _full_spec((1, 128))],
        out_specs=[_row_spec(H), _row_spec(128)],
        out_shape=[jax.ShapeDtypeStruct((NP, H), jnp.float32),
                   jax.ShapeDtypeStruct((NP, 128), jnp.float32)],
    )(g, bias, gam, bet, gw, gb)


def _pool_tc(x, gate3, batch3):
    return pl.pallas_call(
        _pool_body,
        grid=(GRID,),
        in_specs=[_row_spec(H),
                  pl.BlockSpec((1, 1, BM), lambda i: (i, 0, 0)),
                  pl.BlockSpec((1, 1, BM), lambda i: (i, 0, 0))],
        out_specs=[_full_spec((B, H)), _full_spec((B, 128))],
        out_shape=[jax.ShapeDtypeStruct((B, H), jnp.float32),
                   jax.ShapeDtypeStruct((B, 128), jnp.float32)],
    )(x, gate3, batch3)


def _mlp_tc(u, den, layers, ow, ob):
    flat = []
    for lw, lb, lg, lt in layers:
        flat += [lw, lb, lg, lt]
    specs = [_full_spec((B, H)), _full_spec((B, 128))]
    for _ in range(5):
        specs += [_full_spec((H, H)), _full_spec((1, H)),
                  _full_spec((1, H)), _full_spec((1, H))]
    specs += [_full_spec((H, 128)), _full_spec((1, 128))]
    return pl.pallas_call(
        _mlp_body,
        grid=(1,),
        in_specs=specs,
        out_specs=[_full_spec((B, 128))],
        out_shape=[jax.ShapeDtypeStruct((B, 128), jnp.float32)],
    )(u, den, *flat, ow, ob)


def _edge_body(h_hbm, asv_hbm, adv_hbm, src_hbm, dst_hbm, rp_hbm, z_hbm,
               out_hbm,
               acc, stage_a, stage_b, srcb, dstb, asb, adb, rpb, den, albuf,
               sem_a, sem_b, sem_ra, sem_rb):
    wid = lax.axis_index("s") * 2 + lax.axis_index("c")
    wid = pl.multiple_of(wid, 1)

    def edge_vals(t, c0, e_lo, e_hi, base):
        sl = pl.ds(t * 16, 16)
        gid = c0 + t * 16 + lax.iota(jnp.int32, 16)
        mask = (gid >= e_lo) & (gid < e_hi)
        s = asb[sl] + adb[sl]
        e = jnp.maximum(s, 0.2 * s)
        ee = jnp.exp(jnp.minimum(e, 60.0))
        eem = jnp.where(mask, ee, 0.0)
        dl = jnp.clip(dstb[sl] - base, 0, RPW - 1)
        return eem, dl

    def load_chunk(c0):
        c0 = pl.multiple_of(c0, 8)
        cp1 = pltpu.make_async_copy(src_hbm.at[pl.ds(c0, CH)], srcb, sem_a)
        cp2 = pltpu.make_async_copy(dst_hbm.at[pl.ds(c0, CH)], dstb, sem_b)
        cp1.start(); cp2.start()
        cp1.wait(); cp2.wait()
        gathers = []
        for q in range(CH // 128):
            qs = pl.ds(q * 128, 128)
            gathers.append(pltpu.make_async_copy(
                asv_hbm.at[srcb.at[qs]], asb.at[qs], sem_a))
            gathers.append(pltpu.make_async_copy(
                adv_hbm.at[dstb.at[qs]], adb.at[qs], sem_b))
        for cp in gathers:
            cp.start()
        for cp in gathers:
            cp.wait()

    def one_pass(p, pcarry):
        base = pl.multiple_of((p * NW + wid) * RPW, RPW)

        pltpu.sync_copy(z_hbm, acc)
        for q in range(3):
            den[pl.ds(q * 16, 16)] = jnp.zeros((16,), jnp.float32)

        pltpu.sync_copy(rp_hbm.at[pl.ds(base, 56)], rpb)
        e_lo = rpb[pl.ds(0, 16)][0]
        e_hi = rpb[pl.ds(RPW, 16)][0]
        e_al = e_lo - lax.rem(e_lo, 8)
        nch = lax.shift_right_logical(e_hi - e_al + (CH - 1), 9)

        def sweep1(c, _):
            c0 = e_al + c * CH
            load_chunk(c0)

            def grp(t, _):
                eem, dl = edge_vals(t, c0, e_lo, e_hi, base)
                plsc.addupdate_scatter(den, [dl], eem)
                return _

            lax.fori_loop(0, CH // 16, grp, 0)
            return _

        lax.fori_loop(0, nch, sweep1, 0)

        def fire(t, stage, sem):
            pltpu.make_async_copy(
                h_hbm.at[srcb.at[pl.ds(t * NB, NB)]], stage, sem).start()

        def proc(t, c0, stage, sem):
            dls = []
            for g in range(NB // 16):
                eem, dl = edge_vals(t * (NB // 16) + g, c0, e_lo, e_hi, base)
                denv = plsc.load_gather(den, [dl])
                albuf[pl.ds(g * 16, 16)] = eem / (denv + 1e-16)
                dls.append(dl)
            pltpu.make_async_copy(
                h_hbm.at[srcb.at[pl.ds(t * NB, NB)]], stage, sem).wait()

            for j in range(NB):
                ofs = dls[j // 16][j % 16] * H
                alv = plsc.load_gather(
                    albuf, [jnp.full((16,), j, jnp.int32)])

                @plsc.parallel_loop(0, H // 16, unroll=8)
                def _row(k):
                    k16 = k * 16
                    plsc.addupdate(acc.at[pl.ds(ofs + k16, 16)],
                                   alv * stage[j, pl.ds(k16, 16)])

        NBCH = CH // NB

        def sweep2(c, _):
            c0 = e_al + c * CH
            load_chunk(c0)
            fire(0, stage_a, sem_ra)

            def pair(i, carry):
                t0 = 2 * i
                fire(t0 + 1, stage_b, sem_rb)
                proc(t0, c0, stage_a, sem_ra)

                @pl.when(i < NBCH // 2 - 1)
                def _prefetch():
                    fire(t0 + 2, stage_a, sem_ra)

                proc(t0 + 1, c0, stage_b, sem_rb)
                return carry

            lax.fori_loop(0, NBCH // 2, pair, 0)
            return _

        lax.fori_loop(0, nch, sweep2, 0)

        pltpu.sync_copy(acc, out_hbm.at[pl.ds(pl.multiple_of(base * H, RPW * H),
                                              RPW * H)])
        return pcarry

    lax.fori_loop(0, NPASS, one_pass, 0)


def _edge_sc(h, asv, adv, srcs, dsts, rowptr, zeros):
    mesh = plsc.VectorSubcoreMesh(core_axis_name="c", subcore_axis_name="s",
                                  num_cores=2, num_subcores=16)
    f = pl.kernel(
        _edge_body,
        out_type=jax.ShapeDtypeStruct((NP * H,), jnp.float32),
        mesh=mesh,
        scratch_types=[
            pltpu.VMEM((RPW * H,), jnp.float32),
            pltpu.VMEM((NB, H), jnp.float32),
            pltpu.VMEM((NB, H), jnp.float32),
            pltpu.VMEM((CH,), jnp.int32),
            pltpu.VMEM((CH,), jnp.int32),
            pltpu.VMEM((CH,), jnp.float32),
            pltpu.VMEM((CH,), jnp.float32),
            pltpu.VMEM((56,), jnp.int32),
            pltpu.VMEM((48,), jnp.float32),
            pltpu.VMEM((NB,), jnp.float32),
            pltpu.SemaphoreType.DMA,
            pltpu.SemaphoreType.DMA,
            pltpu.SemaphoreType.DMA,
            pltpu.SemaphoreType.DMA,
        ],
        compiler_params=pltpu.CompilerParams(needs_layout_passes=False),
    )
    return f(h, asv, adv, srcs, dsts, rowptr, zeros)


def kernel(tree_x, edge_index, batch, params):
    f32 = jnp.float32

    loops = jnp.arange(N, dtype=jnp.int32)
    src0 = jnp.concatenate([edge_index[0].astype(jnp.int32), loops])
    dst0 = jnp.concatenate([edge_index[1].astype(jnp.int32), loops])
    key = (dst0.astype(jnp.uint32) << 18) | jnp.arange(EE, dtype=jnp.uint32)
    skey = lax.sort(key, is_stable=False)
    srcs = jnp.zeros((EP,), jnp.int32).at[:EE].set(
        src0[(skey & 0x3FFFF).astype(jnp.int32)])
    dsts = jnp.zeros((EP,), jnp.int32).at[:EE].set(
        (skey >> 18).astype(jnp.int32))
    rowptr = jnp.searchsorted(dsts[:EE], jnp.arange(NP + 1, dtype=jnp.int32),
                              side="left").astype(jnp.int32)
    rowptr = jnp.pad(rowptr, (0, 10256 - (NP + 1)), constant_values=EE)

    batch_p = jnp.full((NP,), B, jnp.int32).at[:N].set(batch.astype(jnp.int32))
    batch3 = batch_p.reshape(GRID, 1, BM)

    x0 = jnp.zeros((NP, 32), f32).at[:N, :17].set(tree_x)
    zeros = jnp.zeros((RPW * H,), f32)

    def avec(i):
        a = jnp.zeros((H, 128), f32)
        a = a.at[:, 0].set(params[f"gat{i}_asrc"])
        return a.at[:, 1].set(params[f"gat{i}_adst"])

    def row(v):
        return v.reshape(1, -1).astype(f32)

    w0 = jnp.zeros((32, H), f32).at[:17].set(params["gat0_W"])
    h, asad = _gat_mm0(x0, w0, avec(0))
    g = _edge_sc(h, asad[:, 0], asad[:, 1], srcs, dsts, rowptr, zeros)
    g = g.reshape(NP, H)
    for i in range(1, 5):
        h, asad = _gat_mm(g, row(params[f"gat{i - 1}_b"]),
                          row(params[f"gnorm{i - 1}_g"]),
                          row(params[f"gnorm{i - 1}_b"]),
                          params[f"gat{i}_W"], avec(i))
        g = _edge_sc(h, asad[:, 0], asad[:, 1], srcs, dsts, rowptr, zeros)
        g = g.reshape(NP, H)

    gw = jnp.zeros((H, 128), f32).at[:, 0].set(params["gate_W"][:, 0])
    gb = jnp.zeros((1, 128), f32).at[0, 0].set(params["gate_b"][0])
    x5, gate = _final_tc(g, row(params["gat4_b"]), row(params["gnorm4_g"]),
                         row(params["gnorm4_b"]), gw, gb)
    gate3 = gate[:, 0].reshape(GRID, 1, BM)
    u, den = _pool_tc(x5, gate3, batch3)

    layers = [(params[f"fcn{i}_W"], row(params[f"fcn{i}_b"]),
               row(params[f"fnorm{i}_g"]), row(params[f"fnorm{i}_b"]))
              for i in range(5)]
    ow = jnp.zeros((H, 128), f32).at[:, 0].set(params["out_W"][:, 0])
    ob = jnp.zeros((1, 128), f32).at[0, 0].set(params["out_b"][0])
    out, = _mlp_tc(u, den, layers, ow, ob)
    return out[:, :1]
